# baseline (speedup 1.0000x reference)
import functools

import jax
import jax.numpy as jnp
from jax import lax
from jax.experimental import pallas as pl
from jax.experimental.pallas import tpu as pltpu
from jax.experimental.pallas import tpu_sc as plsc

VOCAB = 1000000
EMB = 64
B = 4096
L = 200

NC = 2
NS = 16
NW = NC * NS
RPW = B // NW
HALF = L // 2

_mesh = plsc.VectorSubcoreMesh(
    core_axis_name="c", subcore_axis_name="s", num_cores=NC, num_subcores=NS
)


@functools.partial(
    pl.kernel,
    out_type=jax.ShapeDtypeStruct((B, EMB), jnp.float32),
    mesh=_mesh,
    scratch_types=[
        pltpu.VMEM((2 * RPW, HALF), jnp.int32),
        pltpu.VMEM((L, EMB), jnp.float32),
        pltpu.VMEM((RPW, EMB), jnp.float32),
        pltpu.SemaphoreType.DMA,
    ],
    compiler_params=pltpu.CompilerParams(use_tc_tiling_on_sc=False),
)
def _encode(ids_hbm, table_hbm, out_hbm, ids_v, emb_v, out_v, sem):
    wid = lax.axis_index("s") * NC + lax.axis_index("c")
    base = wid * RPW

    pltpu.sync_copy(ids_hbm.at[pl.ds(base * 2, 2 * RPW)], ids_v)

    inv_l = jnp.full((16,), 1.0 / L, dtype=jnp.float32)

    def row_body(r, carry):
        c1 = pltpu.async_copy(table_hbm.at[ids_v.at[2 * r]],
                              emb_v.at[pl.ds(0, HALF)], sem)
        c2 = pltpu.async_copy(table_hbm.at[ids_v.at[2 * r + 1]],
                              emb_v.at[pl.ds(HALF, HALF)], sem)
        c1.wait()
        c2.wait()

        def acc_body(l, acc):
            a0, a1, a2, a3 = acc
            return (
                a0 + emb_v[l, pl.ds(0, 16)],
                a1 + emb_v[l, pl.ds(16, 16)],
                a2 + emb_v[l, pl.ds(32, 16)],
                a3 + emb_v[l, pl.ds(48, 16)],
            )

        z = jnp.zeros((16,), jnp.float32)
        a0, a1, a2, a3 = lax.fori_loop(0, L, acc_body, (z, z, z, z))
        out_v[r, pl.ds(0, 16)] = a0 * inv_l
        out_v[r, pl.ds(16, 16)] = a1 * inv_l
        out_v[r, pl.ds(32, 16)] = a2 * inv_l
        out_v[r, pl.ds(48, 16)] = a3 * inv_l
        return carry

    lax.fori_loop(0, RPW, row_body, 0)

    pltpu.sync_copy(out_v, out_hbm.at[pl.ds(base, RPW)])


def kernel(token_ids, table):
    ids2 = token_ids.astype(jnp.int32).reshape(2 * B, HALF)
    return _encode(ids2, table)


# speedup vs baseline: 1.2413x; 1.2413x over previous
import functools

import jax
import jax.numpy as jnp
from jax import lax
from jax.experimental import pallas as pl
from jax.experimental.pallas import tpu as pltpu
from jax.experimental.pallas import tpu_sc as plsc

VOCAB = 1000000
EMB = 64
B = 4096
L = 200

NC = 2
NS = 16
NW = NC * NS
RPW = B // NW
HALF = L // 2
NBUF = 4

_mesh = plsc.VectorSubcoreMesh(
    core_axis_name="c", subcore_axis_name="s", num_cores=NC, num_subcores=NS
)


@functools.partial(
    pl.kernel,
    out_type=jax.ShapeDtypeStruct((B, EMB), jnp.float32),
    mesh=_mesh,
    scratch_types=[
        pltpu.VMEM((2 * RPW, HALF), jnp.int32),
        pltpu.VMEM((NBUF, L, EMB), jnp.float32),
        pltpu.VMEM((RPW, EMB), jnp.float32),
        [pltpu.SemaphoreType.DMA] * NBUF,
    ],
    compiler_params=pltpu.CompilerParams(use_tc_tiling_on_sc=False),
)
def _encode(ids_hbm, table_hbm, out_hbm, ids_v, emb_v, out_v, sems):
    wid = lax.axis_index("s") * NC + lax.axis_index("c")
    base = wid * RPW

    pltpu.sync_copy(ids_hbm.at[pl.ds(base * 2, 2 * RPW)], ids_v)

    inv_l = jnp.full((16,), 1.0 / L, dtype=jnp.float32)

    def fire(r, b):
        pltpu.async_copy(table_hbm.at[ids_v.at[2 * r]],
                         emb_v.at[b, pl.ds(0, HALF)], sems[b])
        pltpu.async_copy(table_hbm.at[ids_v.at[2 * r + 1]],
                         emb_v.at[b, pl.ds(HALF, HALF)], sems[b])

    def drain(b):
        pltpu.make_async_copy(table_hbm.at[ids_v.at[0]],
                              emb_v.at[b, pl.ds(0, HALF)], sems[b]).wait()
        pltpu.make_async_copy(table_hbm.at[ids_v.at[1]],
                              emb_v.at[b, pl.ds(HALF, HALF)], sems[b]).wait()

    def accumulate(r, b):
        def acc_body(i, acc):
            a0, a1, a2, a3 = acc
            l = 4 * i
            for u in range(4):
                a0 = a0 + emb_v[b, l + u, pl.ds(0, 16)]
                a1 = a1 + emb_v[b, l + u, pl.ds(16, 16)]
                a2 = a2 + emb_v[b, l + u, pl.ds(32, 16)]
                a3 = a3 + emb_v[b, l + u, pl.ds(48, 16)]
            return (a0, a1, a2, a3)

        z = jnp.zeros((16,), jnp.float32)
        a0, a1, a2, a3 = lax.fori_loop(0, L // 4, acc_body, (z, z, z, z))
        out_v[r, pl.ds(0, 16)] = a0 * inv_l
        out_v[r, pl.ds(16, 16)] = a1 * inv_l
        out_v[r, pl.ds(32, 16)] = a2 * inv_l
        out_v[r, pl.ds(48, 16)] = a3 * inv_l

    for b in range(NBUF):
        fire(b, b)

    def group_body(g, carry):
        r0 = NBUF * g
        for b in range(NBUF):
            r = r0 + b
            drain(b)
            accumulate(r, b)
            fire(jnp.minimum(r + NBUF, RPW - 1), b)
        return carry

    lax.fori_loop(0, RPW // NBUF - 1, group_body, 0)

    r_last = RPW - NBUF
    for b in range(NBUF):
        drain(b)
        accumulate(r_last + b, b)

    pltpu.sync_copy(out_v, out_hbm.at[pl.ds(base, RPW)])


def kernel(token_ids, table):
    ids2 = token_ids.astype(jnp.int32).reshape(2 * B, HALF)
    return _encode(ids2, table)


# speedup vs baseline: 1.2431x; 1.0015x over previous
import functools

import jax
import jax.numpy as jnp
from jax import lax
from jax.experimental import pallas as pl
from jax.experimental.pallas import tpu as pltpu
from jax.experimental.pallas import tpu_sc as plsc

VOCAB = 1000000
EMB = 64
B = 4096
L = 200

NC = 2
NS = 16
NW = NC * NS
RPW = B // NW
H0 = 104
H1 = L - H0
NBUF = 4

_mesh = plsc.VectorSubcoreMesh(
    core_axis_name="c", subcore_axis_name="s", num_cores=NC, num_subcores=NS
)


@functools.partial(
    pl.kernel,
    out_type=jax.ShapeDtypeStruct((B, EMB), jnp.float32),
    mesh=_mesh,
    scratch_types=[
        pltpu.VMEM((RPW, H0), jnp.int32),
        pltpu.VMEM((RPW, H1), jnp.int32),
        pltpu.VMEM((NBUF, L, EMB), jnp.float32),
        pltpu.VMEM((RPW, EMB), jnp.float32),
        [pltpu.SemaphoreType.DMA] * NBUF,
    ],
    compiler_params=pltpu.CompilerParams(use_tc_tiling_on_sc=False),
)
def _encode(ids_hbm, table_hbm, out_hbm, ids0_v, ids1_v, emb_v, out_v, sems):
    wid = lax.axis_index("s") * NC + lax.axis_index("c")
    base = wid * RPW

    pltpu.sync_copy(ids_hbm.at[pl.ds(base, RPW), pl.ds(0, H0)], ids0_v)
    pltpu.sync_copy(ids_hbm.at[pl.ds(base, RPW), pl.ds(H0, H1)], ids1_v)

    inv_l = jnp.full((16,), 1.0 / L, dtype=jnp.float32)

    def fire(r, b):
        pltpu.async_copy(table_hbm.at[ids0_v.at[r]],
                         emb_v.at[b, pl.ds(0, H0)], sems[b])
        pltpu.async_copy(table_hbm.at[ids1_v.at[r]],
                         emb_v.at[b, pl.ds(H0, H1)], sems[b])

    def drain(b):
        pltpu.make_async_copy(table_hbm.at[ids0_v.at[0]],
                              emb_v.at[b, pl.ds(0, H0)], sems[b]).wait()
        pltpu.make_async_copy(table_hbm.at[ids1_v.at[0]],
                              emb_v.at[b, pl.ds(H0, H1)], sems[b]).wait()

    def accumulate(r, b):
        def acc_body(i, acc):
            a0, a1, a2, a3 = acc
            l = 4 * i
            for u in range(4):
                a0 = a0 + emb_v[b, l + u, pl.ds(0, 16)]
                a1 = a1 + emb_v[b, l + u, pl.ds(16, 16)]
                a2 = a2 + emb_v[b, l + u, pl.ds(32, 16)]
                a3 = a3 + emb_v[b, l + u, pl.ds(48, 16)]
            return (a0, a1, a2, a3)

        z = jnp.zeros((16,), jnp.float32)
        a0, a1, a2, a3 = lax.fori_loop(0, L // 4, acc_body, (z, z, z, z))
        out_v[r, pl.ds(0, 16)] = a0 * inv_l
        out_v[r, pl.ds(16, 16)] = a1 * inv_l
        out_v[r, pl.ds(32, 16)] = a2 * inv_l
        out_v[r, pl.ds(48, 16)] = a3 * inv_l

    for b in range(NBUF):
        fire(b, b)

    def group_body(g, carry):
        r0 = NBUF * g
        for b in range(NBUF):
            r = r0 + b
            drain(b)
            accumulate(r, b)
            fire(r + NBUF, b)
        return carry

    lax.fori_loop(0, RPW // NBUF - 1, group_body, 0)

    r_last = RPW - NBUF
    for b in range(NBUF):
        drain(b)
        accumulate(r_last + b, b)

    pltpu.sync_copy(out_v, out_hbm.at[pl.ds(base, RPW)])


def kernel(token_ids, table):
    return _encode(token_ids, table)
